# baseline (speedup 1.0000x reference)
import functools

import jax
import jax.numpy as jnp
from jax import lax
from jax.experimental import pallas as pl
from jax.experimental.pallas import tpu as pltpu
from jax.experimental.pallas import tpu_sc as plsc

NUM_CORES = 2
NUM_SUBCORES = 16
NW = NUM_CORES * NUM_SUBCORES
LANES = 16
BATCH = 16384
DIM = 32
BPW = BATCH // NW
CHUNK = 128
NCHUNK = BPW // CHUNK
GROUPS = BPW // LANES


def _sc_body(heads, rels, tails, eh, et, rf, ri, out,
             hidx, ridx, tidx, gh1, gr1, gt1, gh2, gr2, gt2, qt, outv, sem):
    c = lax.axis_index("c")
    s = lax.axis_index("s")
    wid = s * NUM_CORES + c
    base = wid * BPW

    pltpu.sync_copy(heads.at[pl.ds(base, BPW)], hidx)
    pltpu.sync_copy(rels.at[pl.ds(base, BPW)], ridx)
    pltpu.sync_copy(tails.at[pl.ds(base, BPW)], tidx)

    copies = []
    for ch in range(NCHUNK):
        d = pl.ds(ch * CHUNK, CHUNK)
        copies.append(pltpu.async_copy(eh.at[hidx.at[d]], gh1.at[d], sem))
        copies.append(pltpu.async_copy(rf.at[ridx.at[d]], gr1.at[d], sem))
        copies.append(pltpu.async_copy(et.at[tidx.at[d]], gt1.at[d], sem))
        copies.append(pltpu.async_copy(eh.at[tidx.at[d]], gh2.at[d], sem))
        copies.append(pltpu.async_copy(ri.at[ridx.at[d]], gr2.at[d], sem))
        copies.append(pltpu.async_copy(et.at[hidx.at[d]], gt2.at[d], sem))
    for cp in copies:
        cp.wait()

    lane_iota = lax.iota(jnp.int32, LANES)

    def row_body(i, carry):
        lo = pl.ds(0, LANES)
        hi = pl.ds(LANES, LANES)
        pa = gh1[i, lo] * gr1[i, lo] * gt1[i, lo] + gh2[i, lo] * gr2[i, lo] * gt2[i, lo]
        pb = gh1[i, hi] * gr1[i, hi] * gt1[i, hi] + gh2[i, hi] * gr2[i, hi] * gt2[i, hi]
        q = pa + pb
        g = i // LANES
        l = i - g * LANES
        flat_idx = g * (LANES * LANES) + lane_iota * LANES + l
        plsc.store_scatter(qt, [flat_idx], q)
        return carry

    lax.fori_loop(0, BPW, row_body, 0)

    def grp_body(g, carry):
        gbase = g * (LANES * LANES)
        acc = qt[pl.ds(gbase, LANES)]
        for dd in range(1, LANES):
            acc = acc + qt[pl.ds(gbase + dd * LANES, LANES)]
        outv[pl.ds(g * LANES, LANES)] = acc * 0.5
        return carry

    lax.fori_loop(0, GROUPS, grp_body, 0)

    pltpu.sync_copy(outv, out.at[pl.ds(base, BPW)])


@jax.jit
def kernel(heads, rels, tails, ent_embs_h, ent_embs_t, rel_embs_f, rel_embs_i):
    heads = heads.astype(jnp.int32)
    rels = rels.astype(jnp.int32)
    tails = tails.astype(jnp.int32)

    mesh = plsc.VectorSubcoreMesh(
        core_axis_name="c", subcore_axis_name="s",
        num_cores=NUM_CORES, num_subcores=NUM_SUBCORES)

    run = pl.kernel(
        _sc_body,
        out_type=jax.ShapeDtypeStruct((BATCH,), jnp.float32),
        mesh=mesh,
        scratch_types=[
            pltpu.VMEM((BPW,), jnp.int32),
            pltpu.VMEM((BPW,), jnp.int32),
            pltpu.VMEM((BPW,), jnp.int32),
            pltpu.VMEM((BPW, DIM), jnp.float32),
            pltpu.VMEM((BPW, DIM), jnp.float32),
            pltpu.VMEM((BPW, DIM), jnp.float32),
            pltpu.VMEM((BPW, DIM), jnp.float32),
            pltpu.VMEM((BPW, DIM), jnp.float32),
            pltpu.VMEM((BPW, DIM), jnp.float32),
            pltpu.VMEM((GROUPS * LANES * LANES,), jnp.float32),
            pltpu.VMEM((BPW,), jnp.float32),
            pltpu.SemaphoreType.DMA,
        ],
        compiler_params=pltpu.CompilerParams(
            needs_layout_passes=False, use_tc_tiling_on_sc=False),
        name="simple_score_sc",
    )
    return run(heads, rels, tails, ent_embs_h, ent_embs_t,
               rel_embs_f, rel_embs_i)
